# baseline (speedup 1.0000x reference)
import dataclasses
import functools

import jax
import jax.numpy as jnp
from jax import lax
from jax.experimental import pallas as pl
from jax.experimental.pallas import tpu as pltpu
from jax.experimental.pallas import tpu_sc as plsc

N = 10000
E = 160000
F_IN = 256
HID = 256
F_OUT = 128

NS = 16
E_PAD = 163840
CHUNK = 80
EDGES_PER_W32 = E // 32
ACC_ROWS = 10016
DUMMY = N
ROWS_PER_TILE = N // NS
BLK = 2000

_f32 = jnp.float32


def _vsmesh():
    return plsc.VectorSubcoreMesh(core_axis_name="c", subcore_axis_name="s")


def _sc_compiler_params(layout_passes=True):
    return pltpu.CompilerParams(
        use_tc_tiling_on_sc=False,
        needs_layout_passes=layout_passes,
        internal_scratch_in_bytes=0,
    )


def _deg_call(edge_index):
    nfull = EDGES_PER_W32 // 16
    rem = EDGES_PER_W32 - nfull * 16

    @functools.partial(
        pl.kernel,
        out_type=jax.ShapeDtypeStruct((32, N), _f32),
        mesh=_vsmesh(),
        scratch_types=[
            pltpu.VMEM((EDGES_PER_W32 + 16,), jnp.int32),
            pltpu.VMEM((10016,), _f32),
        ],
        compiler_params=_sc_compiler_params(layout_passes=False),
    )
    def deg_kernel(edges_hbm, out_hbm, dstv, histv):
        c = lax.axis_index("c")
        s = lax.axis_index("s")
        w = c * NS + s
        dstv[pl.ds(EDGES_PER_W32 - rem, 16)] = jnp.zeros((16,), jnp.int32)
        pltpu.sync_copy(edges_hbm.at[1].at[pl.ds(w * EDGES_PER_W32,
                                                 EDGES_PER_W32)],
                        dstv.at[pl.ds(0, EDGES_PER_W32)])
        zf = jnp.zeros((16,), _f32)
        onef = jnp.ones((16,), _f32)

        @pl.loop(0, 10016 // 16)
        def _(i):
            histv[pl.ds(i * 16, 16)] = zf

        @pl.loop(0, nfull)
        def _(i):
            idx = dstv[pl.ds(i * 16, 16)]
            plsc.addupdate_scatter(histv, [idx], onef)

        tail = dstv[pl.ds(nfull * 16, 16)]
        lane = lax.broadcasted_iota(jnp.int32, (16,), 0)
        plsc.addupdate_scatter(histv, [tail], onef, mask=lane < rem)

        pltpu.sync_copy(histv.at[pl.ds(0, N)], out_hbm.at[w])

    return deg_kernel(edge_index)


def _agg_call(hs2, ed_t, feature_split):
    nt, nchunk, _two, chunk = ed_t.shape
    f = hs2.shape[-1]
    zslices = (ACC_ROWS // NS) // chunk
    zrem = (ACC_ROWS // NS) - zslices * chunk
    assert nchunk % 8 == 0

    @functools.partial(
        pl.kernel,
        out_type=jax.ShapeDtypeStruct((2, N, f), _f32),
        mesh=_vsmesh(),
        scratch_types=[
            pltpu.VMEM((16, chunk), jnp.int32),
            pltpu.VMEM((chunk, f), _f32),
            pltpu.VMEM((chunk, f), _f32),
            pltpu.VMEM((chunk, f), _f32),
            pltpu.VMEM((chunk, f), _f32),
            pltpu.VMEM_SHARED((ACC_ROWS, f), _f32),
        ] + [pltpu.SemaphoreType.DMA] * 16,
        compiler_params=_sc_compiler_params(),
    )
    def agg_kernel(hs_hbm, ed_hbm, out_hbm, idxv, r0, r1, r2, r3, acc,
                   *sems):
        gs = sems[0:4]
        ss = sems[4:8]
        ia = sems[8:12]
        ib = sems[12:16]
        rbufs = (r0, r1, r2, r3)
        c = lax.axis_index("c")
        s = lax.axis_index("s")
        if feature_split:
            hs = hs_hbm.at[c]
            row = s
        else:
            hs = hs_hbm
            row = c * NS + s
        ed = ed_hbm.at[row]
        out = out_hbm.at[c]

        def fetch(j, p, k, sem):
            pltpu.async_copy(ed.at[j], idxv.at[pl.ds(8 * p + 2 * k, 2)], sem)

        def fetch_wait(p, k, sem):
            pltpu.make_async_copy(
                ed.at[0], idxv.at[pl.ds(8 * p + 2 * k, 2)], sem).wait()

        def gather(p, k):
            pltpu.async_copy(hs.at[idxv.at[8 * p + 2 * k]], rbufs[k], gs[k])

        def gather_wait(p, k):
            pltpu.make_async_copy(
                hs.at[idxv.at[8 * p + 2 * k]], rbufs[k], gs[k]).wait()

        def scatter(p, k):
            pltpu.async_copy(rbufs[k], acc.at[idxv.at[8 * p + 2 * k + 1]],
                             ss[k], add=True)

        def scatter_wait(p, k):
            pltpu.make_async_copy(
                rbufs[k], acc.at[idxv.at[8 * p + 2 * k + 1]], ss[k]).wait()

        zf = jnp.zeros((16,), _f32)

        @pl.loop(0, chunk)
        def _(r):
            @pl.loop(0, f // 16)
            def _(q):
                r0[r, pl.ds(q * 16, 16)] = zf

        base = s * (ACC_ROWS // NS)

        @pl.loop(0, zslices)
        def _(k):
            pltpu.sync_copy(r0, acc.at[pl.ds(base + k * chunk, chunk)])

        pltpu.sync_copy(r0.at[pl.ds(0, zrem)],
                        acc.at[pl.ds(base + zslices * chunk, zrem)])

        plsc.subcore_barrier()

        for k in range(4):
            fetch(k, 0, k, ia[k])
            fetch(4 + k, 1, k, ib[k])
        for k in range(4):
            fetch_wait(0, k, ia[k])
            gather(0, k)

        @pl.loop(0, nchunk // 8)
        def _(i):
            cA = 8 * i
            cB = cA + 4
            for k in range(4):
                gather_wait(0, k)
                scatter(0, k)
            for k in range(4):
                scatter_wait(0, k)
                fetch_wait(1, k, ib[k])
                gather(1, k)

                @pl.when(cA + 8 + k < nchunk)
                def _():
                    fetch(cA + 8 + k, 0, k, ia[k])

            for k in range(4):
                gather_wait(1, k)
                scatter(1, k)
            for k in range(4):
                scatter_wait(1, k)

                @pl.when(cA + 8 + k < nchunk)
                def _():
                    fetch_wait(0, k, ia[k])
                    gather(0, k)

                @pl.when(cB + 8 + k < nchunk)
                def _():
                    fetch(cB + 8 + k, 1, k, ib[k])

        plsc.subcore_barrier()
        pltpu.sync_copy(acc.at[pl.ds(s * ROWS_PER_TILE, ROWS_PER_TILE)],
                        out.at[pl.ds(s * ROWS_PER_TILE, ROWS_PER_TILE)])

    return agg_kernel(hs2, ed_t)


_DOT = functools.partial(
    lax.dot_general,
    precision=lax.Precision.DEFAULT,
    preferred_element_type=_f32,
)


def _mmscale_body(dv_ref, x_ref, w_ref, o_ref):
    hs = _DOT(x_ref[...], w_ref[...], (((1,), (0,)), ((), ()))) * dv_ref[...]
    o_ref[0] = hs[:, :HID // 2]
    o_ref[1] = hs[:, HID // 2:]


def _mmscale_call(dinv, x, w):
    return pl.pallas_call(
        _mmscale_body,
        grid=(N // BLK,),
        in_specs=[pl.BlockSpec((BLK, 1), lambda i: (i, 0)),
                  pl.BlockSpec((BLK, F_IN), lambda i: (i, 0)),
                  pl.BlockSpec((F_IN, HID), lambda i: (0, 0))],
        out_specs=pl.BlockSpec((2, BLK, HID // 2), lambda i: (0, i, 0)),
        out_shape=jax.ShapeDtypeStruct((2, N, HID // 2), _f32),
    )(dinv, x, w)


def _dinv_body(p_ref, dv_ref):
    ones = jnp.ones((32, 1), _f32)
    deg = _DOT(p_ref[...], ones, (((0,), (0,)), ((), ()))) + 1.0
    dv_ref[...] = lax.rsqrt(deg)


def _dinv_call(partials):
    return pl.pallas_call(
        _dinv_body,
        in_specs=[pl.BlockSpec((32, N), lambda: (0, 0))],
        out_specs=pl.BlockSpec((N, 1), lambda: (0, 0)),
        out_shape=jax.ShapeDtypeStruct((N, 1), _f32),
    )(partials)


def _layer_body(lo_ref, hi_ref, hs1lo_ref, hs1hi_ref, dv_ref, b1_ref,
                w2_ref, o2_ref):
    s1 = jnp.concatenate([lo_ref[0] + hs1lo_ref[0],
                          hi_ref[0] + hs1hi_ref[0]], axis=1)
    dinv = dv_ref[...]
    out1 = dinv * s1 + b1_ref[...]
    h = jnp.maximum(out1, 0.0)
    h2 = _DOT(h, w2_ref[...], (((1,), (0,)), ((), ())))
    o2_ref[...] = dinv * h2


def _layer_call(s1, hs1, dinv, b1, w2):
    return pl.pallas_call(
        _layer_body,
        grid=(N // BLK,),
        in_specs=[pl.BlockSpec((1, BLK, HID // 2), lambda i: (0, i, 0)),
                  pl.BlockSpec((1, BLK, HID // 2), lambda i: (1, i, 0)),
                  pl.BlockSpec((1, BLK, HID // 2), lambda i: (0, i, 0)),
                  pl.BlockSpec((1, BLK, HID // 2), lambda i: (1, i, 0)),
                  pl.BlockSpec((BLK, 1), lambda i: (i, 0)),
                  pl.BlockSpec((1, HID), lambda i: (0, 0)),
                  pl.BlockSpec((HID, F_OUT), lambda i: (0, 0))],
        out_specs=pl.BlockSpec((BLK, F_OUT), lambda i: (i, 0)),
        out_shape=jax.ShapeDtypeStruct((N, F_OUT), _f32),
    )(s1, s1, hs1, hs1, dinv, b1, w2)


def _final_body(lo_ref, hi_ref, hs2_ref, dv_ref, b2_ref, o_ref):
    s2 = lo_ref[0] + hi_ref[0] + hs2_ref[...]
    o_ref[...] = dv_ref[...] * s2 + b2_ref[...]


def _final_call(s2, hs2, dinv, b2):
    return pl.pallas_call(
        _final_body,
        grid=(N // BLK,),
        in_specs=[pl.BlockSpec((1, BLK, F_OUT), lambda i: (0, i, 0)),
                  pl.BlockSpec((1, BLK, F_OUT), lambda i: (1, i, 0)),
                  pl.BlockSpec((BLK, F_OUT), lambda i: (i, 0)),
                  pl.BlockSpec((BLK, 1), lambda i: (i, 0)),
                  pl.BlockSpec((1, F_OUT), lambda i: (0, 0))],
        out_specs=pl.BlockSpec((BLK, F_OUT), lambda i: (i, 0)),
        out_shape=jax.ShapeDtypeStruct((N, F_OUT), _f32),
    )(s2, s2, hs2, dinv, b2)


def kernel(x, edge_index, W1, b1, W2, b2):
    pad = E_PAD - E
    src_p = jnp.concatenate([edge_index[0], jnp.zeros((pad,), jnp.int32)])
    dst_p = jnp.concatenate([edge_index[1],
                             jnp.full((pad,), DUMMY, jnp.int32)])
    ed = jnp.stack([src_p.reshape(E_PAD // CHUNK, CHUNK),
                    dst_p.reshape(E_PAD // CHUNK, CHUNK)], axis=1)
    ed1 = ed.reshape(NS, E_PAD // (NS * CHUNK), 2, CHUNK)
    ed2 = ed.reshape(2 * NS, E_PAD // (2 * NS * CHUNK), 2, CHUNK)

    partials = _deg_call(edge_index)
    dinv = _dinv_call(partials)
    hs1 = _mmscale_call(dinv, x, W1)
    s1 = _agg_call(hs1, ed1, True)
    hs2 = _layer_call(s1, hs1, dinv, b1.reshape(1, HID), W2)
    s2 = _agg_call(hs2, ed2, False)
    return _final_call(s2, hs2, dinv, b2.reshape(1, F_OUT))


# speedup vs baseline: 2.4614x; 2.4614x over previous
import dataclasses
import functools

import jax
import jax.numpy as jnp
from jax import lax
from jax.experimental import pallas as pl
from jax.experimental.pallas import tpu as pltpu
from jax.experimental.pallas import tpu_sc as plsc

N = 10000
E = 160000
F_IN = 256
HID = 256
F_OUT = 128

NS = 16
EDGES_PER_W32 = E // 32
ACC_ROWS = 10016
NBUF = 3
ROWS_PER_TILE = N // NS
BLK = 2000

_f32 = jnp.float32


def _vsmesh():
    return plsc.VectorSubcoreMesh(core_axis_name="c", subcore_axis_name="s")


def _sc_compiler_params(layout_passes=True):
    return pltpu.CompilerParams(
        use_tc_tiling_on_sc=False,
        needs_layout_passes=layout_passes,
        internal_scratch_in_bytes=0,
    )


def _deg_call(edge_index):
    nfull = EDGES_PER_W32 // 16
    rem = EDGES_PER_W32 - nfull * 16

    @functools.partial(
        pl.kernel,
        out_type=jax.ShapeDtypeStruct((32, N), _f32),
        mesh=_vsmesh(),
        scratch_types=[
            pltpu.VMEM((EDGES_PER_W32 + 16,), jnp.int32),
            pltpu.VMEM((10016,), _f32),
        ],
        compiler_params=_sc_compiler_params(layout_passes=False),
    )
    def deg_kernel(edges_hbm, out_hbm, dstv, histv):
        c = lax.axis_index("c")
        s = lax.axis_index("s")
        w = c * NS + s
        dstv[pl.ds(EDGES_PER_W32 - rem, 16)] = jnp.zeros((16,), jnp.int32)
        pltpu.sync_copy(edges_hbm.at[1].at[pl.ds(w * EDGES_PER_W32,
                                                 EDGES_PER_W32)],
                        dstv.at[pl.ds(0, EDGES_PER_W32)])
        zf = jnp.zeros((16,), _f32)
        onef = jnp.ones((16,), _f32)

        @pl.loop(0, 10016 // 16)
        def _(i):
            histv[pl.ds(i * 16, 16)] = zf

        @pl.loop(0, nfull)
        def _(i):
            idx = dstv[pl.ds(i * 16, 16)]
            plsc.addupdate_scatter(histv, [idx], onef)

        tail = dstv[pl.ds(nfull * 16, 16)]
        lane = lax.broadcasted_iota(jnp.int32, (16,), 0)
        plsc.addupdate_scatter(histv, [tail], onef, mask=lane < rem)

        pltpu.sync_copy(histv.at[pl.ds(0, N)], out_hbm.at[w])

    return deg_kernel(edge_index)


def _agg_call(hs2, src_t, dst_t, feature_split):
    nt, nchunk, chunk = src_t.shape
    f = hs2.shape[-1]
    zslices = (ACC_ROWS // NS) // chunk
    zrem = (ACC_ROWS // NS) - zslices * chunk

    @functools.partial(
        pl.kernel,
        out_type=jax.ShapeDtypeStruct((2, N, f), _f32),
        mesh=_vsmesh(),
        scratch_types=[
            pltpu.VMEM((nchunk, chunk), jnp.int32),
            pltpu.VMEM((nchunk, chunk), jnp.int32),
            pltpu.VMEM((chunk, f), _f32),
            pltpu.VMEM((chunk, f), _f32),
            pltpu.VMEM((chunk, f), _f32),
            pltpu.VMEM_SHARED((ACC_ROWS, f), _f32),
            pltpu.SemaphoreType.DMA,
            pltpu.SemaphoreType.DMA,
            pltpu.SemaphoreType.DMA,
            pltpu.SemaphoreType.DMA,
            pltpu.SemaphoreType.DMA,
            pltpu.SemaphoreType.DMA,
        ],
        compiler_params=_sc_compiler_params(),
    )
    def agg_kernel(hs_hbm, src_hbm, dst_hbm, out_hbm,
                   srcv, dstv, r0, r1, r2, acc,
                   g0, g1, g2, s0, s1, s2):
        c = lax.axis_index("c")
        s = lax.axis_index("s")
        if feature_split:
            hs = hs_hbm.at[c]
            row = s
        else:
            hs = hs_hbm
            row = c * NS + s
        out = out_hbm.at[c]
        pltpu.sync_copy(src_hbm.at[row], srcv)
        pltpu.sync_copy(dst_hbm.at[row], dstv)

        zf = jnp.zeros((16,), _f32)

        @pl.loop(0, chunk)
        def _(r):
            @pl.loop(0, f // 16)
            def _(q):
                r0[r, pl.ds(q * 16, 16)] = zf

        base = s * (ACC_ROWS // NS)

        @pl.loop(0, zslices)
        def _(k):
            pltpu.sync_copy(r0, acc.at[pl.ds(base + k * chunk, chunk)])

        pltpu.sync_copy(r0.at[pl.ds(0, zrem)],
                        acc.at[pl.ds(base + zslices * chunk, zrem)])

        plsc.subcore_barrier()

        bufs = ((r0, g0, s0), (r1, g1, s1), (r2, g2, s2))[:NBUF]
        nb = len(bufs)
        for k, (r, g, _s) in enumerate(bufs):
            pltpu.async_copy(hs.at[srcv.at[k]], r, g)

        @pl.loop(0, nchunk // nb)
        def _(i):
            j0 = nb * i
            for k, (r, g, ss) in enumerate(bufs):
                j = j0 + k
                pltpu.make_async_copy(hs.at[srcv.at[j]], r, g).wait()
                pltpu.async_copy(r, acc.at[dstv.at[j]], ss, add=True)
            for k, (r, g, ss) in enumerate(bufs):
                j = j0 + k
                pltpu.make_async_copy(r, acc.at[dstv.at[j]], ss).wait()

                @pl.when(j + nb < nchunk)
                def _():
                    pltpu.async_copy(hs.at[srcv.at[j + nb]], r, g)

        for k in range(nchunk % nb):
            j = (nchunk // nb) * nb + k
            r, g, ss = bufs[k]
            pltpu.make_async_copy(hs.at[srcv.at[j]], r, g).wait()
            pltpu.async_copy(r, acc.at[dstv.at[j]], ss, add=True)
        for k in range(nchunk % nb):
            j = (nchunk // nb) * nb + k
            r, g, ss = bufs[k]
            pltpu.make_async_copy(r, acc.at[dstv.at[j]], ss).wait()

        plsc.subcore_barrier()
        pltpu.sync_copy(acc.at[pl.ds(s * ROWS_PER_TILE, ROWS_PER_TILE)],
                        out.at[pl.ds(s * ROWS_PER_TILE, ROWS_PER_TILE)])

    return agg_kernel(hs2, src_t, dst_t)


_DOT = functools.partial(
    lax.dot_general,
    precision=lax.Precision.DEFAULT,
    preferred_element_type=_f32,
)


def _mmscale_body(dv_ref, x_ref, w_ref, o_ref):
    hs = _DOT(x_ref[...], w_ref[...], (((1,), (0,)), ((), ()))) * dv_ref[...]
    o_ref[0] = hs[:, :HID // 2]
    o_ref[1] = hs[:, HID // 2:]


def _mmscale_call(dinv, x, w):
    return pl.pallas_call(
        _mmscale_body,
        grid=(N // BLK,),
        in_specs=[pl.BlockSpec((BLK, 1), lambda i: (i, 0)),
                  pl.BlockSpec((BLK, F_IN), lambda i: (i, 0)),
                  pl.BlockSpec((F_IN, HID), lambda i: (0, 0))],
        out_specs=pl.BlockSpec((2, BLK, HID // 2), lambda i: (0, i, 0)),
        out_shape=jax.ShapeDtypeStruct((2, N, HID // 2), _f32),
    )(dinv, x, w)


def _dinv_body(p_ref, dv_ref):
    ones = jnp.ones((32, 1), _f32)
    deg = _DOT(p_ref[...], ones, (((0,), (0,)), ((), ()))) + 1.0
    dv_ref[...] = lax.rsqrt(deg)


def _dinv_call(partials):
    return pl.pallas_call(
        _dinv_body,
        in_specs=[pl.BlockSpec((32, N), lambda: (0, 0))],
        out_specs=pl.BlockSpec((N, 1), lambda: (0, 0)),
        out_shape=jax.ShapeDtypeStruct((N, 1), _f32),
    )(partials)


def _layer_body(lo_ref, hi_ref, hs1lo_ref, hs1hi_ref, dv_ref, b1_ref,
                w2_ref, o2_ref):
    s1 = jnp.concatenate([lo_ref[0] + hs1lo_ref[0],
                          hi_ref[0] + hs1hi_ref[0]], axis=1)
    dinv = dv_ref[...]
    out1 = dinv * s1 + b1_ref[...]
    h = jnp.maximum(out1, 0.0)
    h2 = _DOT(h, w2_ref[...], (((1,), (0,)), ((), ())))
    o2_ref[...] = dinv * h2


def _layer_call(s1, hs1, dinv, b1, w2):
    return pl.pallas_call(
        _layer_body,
        grid=(N // BLK,),
        in_specs=[pl.BlockSpec((1, BLK, HID // 2), lambda i: (0, i, 0)),
                  pl.BlockSpec((1, BLK, HID // 2), lambda i: (1, i, 0)),
                  pl.BlockSpec((1, BLK, HID // 2), lambda i: (0, i, 0)),
                  pl.BlockSpec((1, BLK, HID // 2), lambda i: (1, i, 0)),
                  pl.BlockSpec((BLK, 1), lambda i: (i, 0)),
                  pl.BlockSpec((1, HID), lambda i: (0, 0)),
                  pl.BlockSpec((HID, F_OUT), lambda i: (0, 0))],
        out_specs=pl.BlockSpec((BLK, F_OUT), lambda i: (i, 0)),
        out_shape=jax.ShapeDtypeStruct((N, F_OUT), _f32),
    )(s1, s1, hs1, hs1, dinv, b1, w2)


def _final_body(lo_ref, hi_ref, hs2_ref, dv_ref, b2_ref, o_ref):
    s2 = lo_ref[0] + hi_ref[0] + hs2_ref[...]
    o_ref[...] = dv_ref[...] * s2 + b2_ref[...]


def _final_call(s2, hs2, dinv, b2):
    return pl.pallas_call(
        _final_body,
        grid=(N // BLK,),
        in_specs=[pl.BlockSpec((1, BLK, F_OUT), lambda i: (0, i, 0)),
                  pl.BlockSpec((1, BLK, F_OUT), lambda i: (1, i, 0)),
                  pl.BlockSpec((BLK, F_OUT), lambda i: (i, 0)),
                  pl.BlockSpec((BLK, 1), lambda i: (i, 0)),
                  pl.BlockSpec((1, F_OUT), lambda i: (0, 0))],
        out_specs=pl.BlockSpec((BLK, F_OUT), lambda i: (i, 0)),
        out_shape=jax.ShapeDtypeStruct((N, F_OUT), _f32),
    )(s2, s2, hs2, dinv, b2)


def kernel(x, edge_index, W1, b1, W2, b2):
    src = edge_index[0]
    dst = edge_index[1]
    src_t1 = src.reshape(NS, 125, 80)
    dst_t1 = dst.reshape(NS, 125, 80)
    src_t2 = src.reshape(2 * NS, 50, 100)
    dst_t2 = dst.reshape(2 * NS, 50, 100)

    partials = _deg_call(edge_index)
    dinv = _dinv_call(partials)
    hs1 = _mmscale_call(dinv, x, W1)
    s1 = _agg_call(hs1, src_t1, dst_t1, True)
    hs2 = _layer_call(s1, hs1, dinv, b1.reshape(1, HID), W2)
    s2 = _agg_call(hs2, src_t2, dst_t2, False)
    return _final_call(s2, hs2, dinv, b2.reshape(1, F_OUT))


# speedup vs baseline: 2.4684x; 1.0028x over previous
import functools

import jax
import jax.numpy as jnp
from jax import lax
from jax.experimental import pallas as pl
from jax.experimental.pallas import tpu as pltpu
from jax.experimental.pallas import tpu_sc as plsc

N = 10000
E = 160000
F_IN = 256
HID = 256
F_OUT = 128

NS = 16
EDGES_PER_W32 = E // 32
ACC_ROWS = 10016
NBUF = 3
ROWS_PER_TILE = N // NS
BLK = 2000

_f32 = jnp.float32


def _vsmesh():
    return plsc.VectorSubcoreMesh(core_axis_name="c", subcore_axis_name="s")


def _sc_compiler_params(layout_passes=True):
    return pltpu.CompilerParams(
        use_tc_tiling_on_sc=False,
        needs_layout_passes=layout_passes,
        internal_scratch_in_bytes=0,
    )


def _deg_call(edge_index):
    nfull = EDGES_PER_W32 // 16
    rem = EDGES_PER_W32 - nfull * 16

    @functools.partial(
        pl.kernel,
        out_type=jax.ShapeDtypeStruct((32, N), _f32),
        mesh=_vsmesh(),
        scratch_types=[
            pltpu.VMEM((EDGES_PER_W32 + 16,), jnp.int32),
            pltpu.VMEM((10016,), _f32),
        ],
        compiler_params=_sc_compiler_params(layout_passes=False),
    )
    def deg_kernel(edges_hbm, out_hbm, dstv, histv):
        c = lax.axis_index("c")
        s = lax.axis_index("s")
        w = c * NS + s
        dstv[pl.ds(EDGES_PER_W32 - rem, 16)] = jnp.zeros((16,), jnp.int32)
        pltpu.sync_copy(edges_hbm.at[1].at[pl.ds(w * EDGES_PER_W32,
                                                 EDGES_PER_W32)],
                        dstv.at[pl.ds(0, EDGES_PER_W32)])
        zf = jnp.zeros((16,), _f32)
        onef = jnp.ones((16,), _f32)

        @pl.loop(0, 10016 // 16)
        def _(i):
            histv[pl.ds(i * 16, 16)] = zf

        @pl.loop(0, nfull)
        def _(i):
            idx = dstv[pl.ds(i * 16, 16)]
            plsc.addupdate_scatter(histv, [idx], onef)

        tail = dstv[pl.ds(nfull * 16, 16)]
        lane = lax.broadcasted_iota(jnp.int32, (16,), 0)
        plsc.addupdate_scatter(histv, [tail], onef, mask=lane < rem)

        pltpu.sync_copy(histv.at[pl.ds(0, N)], out_hbm.at[w])

    return deg_kernel(edge_index)


def _agg_call(hs2, src_t, dst_t, feature_split):
    nt, nchunk, chunk = src_t.shape
    f = hs2.shape[-1]
    zslices = (ACC_ROWS // NS) // chunk
    zrem = (ACC_ROWS // NS) - zslices * chunk

    @functools.partial(
        pl.kernel,
        out_type=jax.ShapeDtypeStruct((2, N, f), _f32),
        mesh=_vsmesh(),
        scratch_types=[
            pltpu.VMEM((nchunk, chunk), jnp.int32),
            pltpu.VMEM((nchunk, chunk), jnp.int32),
            pltpu.VMEM((chunk, f), _f32),
            pltpu.VMEM((chunk, f), _f32),
            pltpu.VMEM((chunk, f), _f32),
            pltpu.VMEM_SHARED((ACC_ROWS, f), _f32),
            pltpu.SemaphoreType.DMA,
            pltpu.SemaphoreType.DMA,
            pltpu.SemaphoreType.DMA,
            pltpu.SemaphoreType.DMA,
            pltpu.SemaphoreType.DMA,
            pltpu.SemaphoreType.DMA,
        ],
        compiler_params=_sc_compiler_params(),
    )
    def agg_kernel(hs_hbm, src_hbm, dst_hbm, out_hbm,
                   srcv, dstv, r0, r1, r2, acc,
                   g0, g1, g2, s0, s1, s2):
        c = lax.axis_index("c")
        s = lax.axis_index("s")
        if feature_split:
            hs = hs_hbm.at[c]
            row = s
        else:
            hs = hs_hbm
            row = c * NS + s
        out = out_hbm.at[c]
        pltpu.sync_copy(src_hbm.at[row], srcv)
        pltpu.sync_copy(dst_hbm.at[row], dstv)

        zf = jnp.zeros((16,), _f32)

        @pl.loop(0, chunk)
        def _(r):
            @pl.loop(0, f // 16)
            def _(q):
                r0[r, pl.ds(q * 16, 16)] = zf

        base = s * (ACC_ROWS // NS)

        @pl.loop(0, zslices)
        def _(k):
            pltpu.sync_copy(r0, acc.at[pl.ds(base + k * chunk, chunk)])

        pltpu.sync_copy(r0.at[pl.ds(0, zrem)],
                        acc.at[pl.ds(base + zslices * chunk, zrem)])

        plsc.subcore_barrier()

        bufs = ((r0, g0, s0), (r1, g1, s1), (r2, g2, s2))[:NBUF]
        nb = len(bufs)
        for k, (r, g, _s) in enumerate(bufs):
            pltpu.async_copy(hs.at[srcv.at[k]], r, g)

        @pl.loop(0, nchunk // nb)
        def _(i):
            j0 = nb * i
            for k, (r, g, ss) in enumerate(bufs):
                j = j0 + k
                pltpu.make_async_copy(hs.at[srcv.at[j]], r, g).wait()
                pltpu.async_copy(r, acc.at[dstv.at[j]], ss, add=True)
            for k, (r, g, ss) in enumerate(bufs):
                j = j0 + k
                pltpu.make_async_copy(r, acc.at[dstv.at[j]], ss).wait()

                @pl.when(j + nb < nchunk)
                def _():
                    pltpu.async_copy(hs.at[srcv.at[j + nb]], r, g)

        for k in range(nchunk % nb):
            j = (nchunk // nb) * nb + k
            r, g, ss = bufs[k]
            pltpu.make_async_copy(hs.at[srcv.at[j]], r, g).wait()
            pltpu.async_copy(r, acc.at[dstv.at[j]], ss, add=True)
        for k in range(nchunk % nb):
            j = (nchunk // nb) * nb + k
            r, g, ss = bufs[k]
            pltpu.make_async_copy(r, acc.at[dstv.at[j]], ss).wait()

        plsc.subcore_barrier()
        pltpu.sync_copy(acc.at[pl.ds(s * ROWS_PER_TILE, ROWS_PER_TILE)],
                        out.at[pl.ds(s * ROWS_PER_TILE, ROWS_PER_TILE)])

    return agg_kernel(hs2, src_t, dst_t)


_DOT = functools.partial(
    lax.dot_general,
    precision=lax.Precision.DEFAULT,
    preferred_element_type=_f32,
)


def _mmscale_body(dv_ref, x_ref, w_ref, o_ref):
    hs = _DOT(x_ref[...], w_ref[...], (((1,), (0,)), ((), ()))) * dv_ref[...]
    o_ref[0] = hs[:, :HID // 2]
    o_ref[1] = hs[:, HID // 2:]


def _mmscale_call(dinv, x, w):
    return pl.pallas_call(
        _mmscale_body,
        grid=(N // BLK,),
        in_specs=[pl.BlockSpec((BLK, 1), lambda i: (i, 0)),
                  pl.BlockSpec((BLK, F_IN), lambda i: (i, 0)),
                  pl.BlockSpec((F_IN, HID), lambda i: (0, 0))],
        out_specs=pl.BlockSpec((2, BLK, HID // 2), lambda i: (0, i, 0)),
        out_shape=jax.ShapeDtypeStruct((2, N, HID // 2), _f32),
    )(dinv, x, w)


def _dinv_body(p_ref, dv_ref):
    ones = jnp.ones((32, 1), _f32)
    deg = _DOT(p_ref[...], ones, (((0,), (0,)), ((), ()))) + 1.0
    dv_ref[...] = lax.rsqrt(deg)


def _dinv_call(partials):
    return pl.pallas_call(
        _dinv_body,
        in_specs=[pl.BlockSpec((32, N), lambda: (0, 0))],
        out_specs=pl.BlockSpec((N, 1), lambda: (0, 0)),
        out_shape=jax.ShapeDtypeStruct((N, 1), _f32),
    )(partials)


def _layer_body(lo_ref, hi_ref, hs1lo_ref, hs1hi_ref, dv_ref, b1_ref,
                w2_ref, o2_ref):
    s1 = jnp.concatenate([lo_ref[0] + hs1lo_ref[0],
                          hi_ref[0] + hs1hi_ref[0]], axis=1)
    dinv = dv_ref[...]
    out1 = dinv * s1 + b1_ref[...]
    h = jnp.maximum(out1, 0.0)
    h2 = _DOT(h, w2_ref[...], (((1,), (0,)), ((), ())))
    o2_ref[...] = dinv * h2


def _layer_call(s1, hs1, dinv, b1, w2):
    return pl.pallas_call(
        _layer_body,
        grid=(N // BLK,),
        in_specs=[pl.BlockSpec((1, BLK, HID // 2), lambda i: (0, i, 0)),
                  pl.BlockSpec((1, BLK, HID // 2), lambda i: (1, i, 0)),
                  pl.BlockSpec((1, BLK, HID // 2), lambda i: (0, i, 0)),
                  pl.BlockSpec((1, BLK, HID // 2), lambda i: (1, i, 0)),
                  pl.BlockSpec((BLK, 1), lambda i: (i, 0)),
                  pl.BlockSpec((1, HID), lambda i: (0, 0)),
                  pl.BlockSpec((HID, F_OUT), lambda i: (0, 0))],
        out_specs=pl.BlockSpec((BLK, F_OUT), lambda i: (i, 0)),
        out_shape=jax.ShapeDtypeStruct((N, F_OUT), _f32),
    )(s1, s1, hs1, hs1, dinv, b1, w2)


def _final_body(lo_ref, hi_ref, hs2_ref, dv_ref, b2_ref, o_ref):
    s2 = lo_ref[0] + hi_ref[0] + hs2_ref[...]
    o_ref[...] = dv_ref[...] * s2 + b2_ref[...]


def _final_call(s2, hs2, dinv, b2):
    return pl.pallas_call(
        _final_body,
        grid=(N // BLK,),
        in_specs=[pl.BlockSpec((1, BLK, F_OUT), lambda i: (0, i, 0)),
                  pl.BlockSpec((1, BLK, F_OUT), lambda i: (1, i, 0)),
                  pl.BlockSpec((BLK, F_OUT), lambda i: (i, 0)),
                  pl.BlockSpec((BLK, 1), lambda i: (i, 0)),
                  pl.BlockSpec((1, F_OUT), lambda i: (0, 0))],
        out_specs=pl.BlockSpec((BLK, F_OUT), lambda i: (i, 0)),
        out_shape=jax.ShapeDtypeStruct((N, F_OUT), _f32),
    )(s2, s2, hs2, dinv, b2)


def kernel(x, edge_index, W1, b1, W2, b2):
    src = edge_index[0]
    dst = edge_index[1]
    src_t1 = src.reshape(NS, 125, 80)
    dst_t1 = dst.reshape(NS, 125, 80)
    src_t2 = src.reshape(2 * NS, 50, 100)
    dst_t2 = dst.reshape(2 * NS, 50, 100)

    partials = _deg_call(edge_index)
    dinv = _dinv_call(partials)
    hs1 = _mmscale_call(dinv, x, W1)
    s1 = _agg_call(hs1, src_t1, dst_t1, True)
    hs2 = _layer_call(s1, hs1, dinv, b1.reshape(1, HID), W2)
    s2 = _agg_call(hs2, src_t2, dst_t2, False)
    return _final_call(s2, hs2, dinv, b2.reshape(1, F_OUT))
